# baseline (speedup 1.0000x reference)
import jax
import jax.numpy as jnp
from jax import lax
from jax.experimental import pallas as pl
from jax.experimental.pallas import tpu as pltpu
from jax.experimental.pallas import tpu_sc as plsc

N_CLS = 100
B = 256
H = 224
W = 224
NW = 32
IMGS_PER_W = B // NW
GROUPS = IMGS_PER_W * 3
NVEC = W // 16
NBUF = 2


def _sc_body(table_hbm, widx_hbm, idx_hbm, clstab_hbm, out_hbm, cls_hbm,
             idx_v, buf, idx16_v, cls_v,
             sem_in0, sem_in1, sem_out0, sem_out1, sem_cls):
    c = lax.axis_index("c")
    s = lax.axis_index("s")
    wid = s * 2 + c
    sems_in = (sem_in0, sem_in1)
    sems_out = (sem_out0, sem_out1)

    @pl.when(wid < 16)
    def _():
        pltpu.sync_copy(idx_hbm.at[pl.ds(wid * 16, 16)], idx16_v)
        pltpu.async_copy(clstab_hbm.at[idx16_v], cls_v, sem_cls).wait()
        pltpu.sync_copy(cls_v, cls_hbm.at[pl.ds(wid * 16, 16)])

    img_base = wid * IMGS_PER_W
    pltpu.sync_copy(widx_hbm.at[wid], idx_v)
    iv = idx_v[pl.ds(0, 16)]

    def compute(b):
        @plsc.parallel_loop(0, H, unroll=2)
        def _row(r):
            for cc in range(NVEC):
                x = buf[b, r, pl.ds(cc * 16, 16)]
                buf[b, r, pl.ds(cc * 16, 16)] = (
                    jnp.minimum(jnp.maximum(x, -0.5), 0.5) + 0.5
                )

    def gather(g, b):
        row = iv[g // 3]
        return pltpu.async_copy(
            table_hbm.at[row, g % 3], buf.at[b], sems_in[b])

    cps_in = [None, None]
    cps_out = [None, None]
    cps_in[0] = gather(0, 0)
    for g in range(GROUPS):
        b = g & 1
        nb = b ^ 1
        if g + 1 < GROUPS:
            if g >= 1:
                cps_out[nb].wait()
            cps_in[nb] = gather(g + 1, nb)
        cps_in[b].wait()
        compute(b)
        cps_out[b] = pltpu.async_copy(
            buf.at[b], out_hbm.at[img_base + g // 3, g % 3], sems_out[b])
    cps_out[0].wait()
    cps_out[1].wait()


@jax.jit
def _run(table, widx, idx, clstab):
    mesh = plsc.VectorSubcoreMesh(
        core_axis_name="c", subcore_axis_name="s", num_cores=2, num_subcores=16
    )
    f = pl.kernel(
        _sc_body,
        out_type=(
            jax.ShapeDtypeStruct((B, 3, H, W), jnp.float32),
            jax.ShapeDtypeStruct((B,), jnp.int32),
        ),
        mesh=mesh,
        scratch_types=[
            pltpu.VMEM((16,), jnp.int32),
            pltpu.VMEM((NBUF, H, W), jnp.float32),
            pltpu.VMEM((16,), jnp.int32),
            pltpu.VMEM((16,), jnp.int32),
            pltpu.SemaphoreType.DMA,
            pltpu.SemaphoreType.DMA,
            pltpu.SemaphoreType.DMA,
            pltpu.SemaphoreType.DMA,
            pltpu.SemaphoreType.DMA,
        ],
    )
    return f(table, widx, idx, clstab)


def kernel(indices, input_tensor, classes_arr):
    idx = indices.astype(jnp.int32)
    widx = jnp.pad(idx.reshape(NW, IMGS_PER_W), ((0, 0), (0, 16 - IMGS_PER_W)))
    clstab = jnp.pad(classes_arr.astype(jnp.int32), (0, 128 - N_CLS))
    imgs, cls = _run(input_tensor, widx, idx, clstab)
    return imgs, cls
